# baseline (speedup 1.0000x reference)
import functools

import jax
import jax.numpy as jnp
import numpy as np
from jax.experimental import pallas as pl

_ROWS = 256
_K = 41
_CH = 16


def _cheb_consts():
    k = np.arange(_K)
    theta = (k + 0.5) * np.pi / _K
    xnodes = np.cos(theta)[None, :]
    j = np.arange(_K)[:, None]
    fit = (2.0 / _K) * np.cos(j * theta[None, :])
    fit[0, :] *= 0.5
    return xnodes.astype(np.float32), fit.astype(np.float32)

_XNODES, _FITM = _cheb_consts()


def _lipswish(t):
    return 0.909 * t * jax.nn.sigmoid(t)


def _dist2_block(xb, xf, hi):
    nb = jnp.sum(xb * xb, axis=1, keepdims=True)
    nf = jnp.sum(xf * xf, axis=1, keepdims=True)
    xb_aug = jnp.concatenate([xb * -2.0, nb, jnp.ones_like(nb)], axis=1)
    xf_aug = jnp.concatenate([xf, jnp.ones_like(nf), nf], axis=1)
    d16 = jax.lax.dot_general(
        xb_aug, xf_aug, (((1,), (1,)), ((), ())),
        preferred_element_type=jnp.float32, precision=hi)
    return jnp.maximum(d16 * 2.0, 0.0)


def _fit_kernel(
    xf_ref, W1c_ref, b1c_ref, We2_ref, b2c_ref, Wg_ref, bg_ref,
    Wc1_ref, b3c_ref, Wc2_ref, bc2_ref, xn_ref, fit_ref,
    cfit_ref,
):
    hi = jax.lax.Precision.HIGHEST
    lo = jax.lax.Precision.DEFAULT
    xf = xf_ref[0]
    dist2 = _dist2_block(xf, xf, hi)
    dmax = jnp.maximum(jnp.max(dist2), 1e-6)

    dn = (xn_ref[...] + 1.0) * (0.5 * dmax)
    m1 = _lipswish(W1c_ref[...] * dn + b1c_ref[...])
    m2 = jax.lax.dot_general(
        We2_ref[...], m1, (((0,), (0,)), ((), ())),
        preferred_element_type=jnp.float32, precision=lo)
    m2 = _lipswish(m2 + b2c_ref[...])
    gate = jax.nn.sigmoid(
        jax.lax.dot_general(
            Wg_ref[...], m2, (((0,), (0,)), ((), ())),
            preferred_element_type=jnp.float32, precision=lo)
        + bg_ref[...])
    h = _lipswish(
        jax.lax.dot_general(
            Wc1_ref[...], m2 * gate, (((0,), (0,)), ((), ())),
            preferred_element_type=jnp.float32, precision=lo)
        + b3c_ref[...])
    wn = jax.lax.dot_general(
        Wc2_ref[...], h, (((0,), (0,)), ((), ())),
        preferred_element_type=jnp.float32, precision=lo) + bc2_ref[...]

    c = jax.lax.dot_general(
        wn, fit_ref[...], (((1,), (1,)), ((), ())),
        preferred_element_type=jnp.float32, precision=hi)
    cfit_ref[0] = jnp.concatenate([c, jnp.full((1, 1), dmax)], axis=1)


def _egnn_block_kernel(
    xb_ref, xf_ref, mrow_ref, mcol_ref, cfit_ref, scale_ref,
    out_ref,
):
    R = xb_ref.shape[1]
    N = xf_ref.shape[1]
    hi = jax.lax.Precision.HIGHEST

    xb = xb_ref[0]
    xf = xf_ref[0]
    mrow = mrow_ref[0]
    mcol = mcol_ref[0]
    scale = scale_ref[0, 0]
    cs = [cfit_ref[0, 0, j] for j in range(_K)]
    dmax = cfit_ref[0, 0, _K]

    dist2 = _dist2_block(xb, xf, hi)

    xs = dist2 * (2.0 / dmax) - 1.0
    chunks = []
    for rc in range(0, R, _CH):
        xsc = xs[rc:rc + _CH]
        xs2c = xsc + xsc
        b1 = jnp.zeros_like(xsc)
        b2 = jnp.zeros_like(xsc)
        for j in range(_K - 1, 0, -1):
            b1, b2 = xs2c * b1 - b2 + cs[j], b1
        chunks.append(xsc * b1 - b2 + cs[0])
    w = jnp.concatenate(chunks, axis=0)
    w = jnp.clip(w, -2.0, 2.0)

    inv_norm = jax.lax.rsqrt(dist2 + 1e-8)
    a = w * mcol * (scale * inv_norm)
    r0 = pl.program_id(1) * R
    col_ids = jax.lax.broadcasted_iota(jnp.int32, (R, N), 1)
    row_ids = jax.lax.broadcasted_iota(jnp.int32, (R, N), 0) + r0
    a = jnp.where(col_ids == row_ids, 0.0, a)

    s = jnp.sum(a, axis=1, keepdims=True)
    t = jax.lax.dot_general(
        a, xf, (((1,), (0,)), ((), ())),
        preferred_element_type=jnp.float32, precision=hi)
    out_ref[0] = (xb + mrow * (s * xb - t)) * mrow


@functools.partial(jax.jit, static_argnames=())
def kernel(x, mask, We1, be1, We2, be2, Wg, bg, Wc1, bc1, Wc2, bc2, scale):
    B, N, D = x.shape
    M = We2.shape[0]
    H = Wc1.shape[1]
    R = _ROWS
    K = _K

    mask_row = mask.reshape(B, N, 1)
    mask_col = mask.reshape(B, 1, N)
    W1c = We1.reshape(M, 1)
    b1c = be1.reshape(M, 1)
    b2c = be2.reshape(M, 1)
    bg2 = bg.reshape(1, 1)
    b3c = bc1.reshape(H, 1)
    bc22 = bc2.reshape(1, 1)
    scale2 = scale.reshape(1, 1)
    xn = jnp.asarray(_XNODES)
    fitm = jnp.asarray(_FITM)

    fullb = lambda shape: pl.BlockSpec(shape, lambda b: (0,) * len(shape))
    cfit = pl.pallas_call(
        _fit_kernel,
        grid=(B,),
        in_specs=[
            pl.BlockSpec((1, N, D), lambda b: (b, 0, 0)),
            fullb((M, 1)), fullb((M, 1)), fullb((M, M)), fullb((M, 1)),
            fullb((M, 1)), fullb((1, 1)), fullb((M, H)), fullb((H, 1)),
            fullb((H, 1)), fullb((1, 1)), fullb((1, K)), fullb((K, K)),
        ],
        out_specs=pl.BlockSpec((1, 1, K + 1), lambda b: (b, 0, 0)),
        out_shape=jax.ShapeDtypeStruct((B, 1, K + 1), jnp.float32),
    )(x, W1c, b1c, We2, b2c, Wg, bg2, Wc1, b3c, Wc2, bc22, xn, fitm)

    grid = (B, N // R)
    full = lambda shape: pl.BlockSpec(shape, lambda b, i: (0,) * len(shape))
    y = pl.pallas_call(
        _egnn_block_kernel,
        grid=grid,
        in_specs=[
            pl.BlockSpec((1, R, D), lambda b, i: (b, i, 0)),
            pl.BlockSpec((1, N, D), lambda b, i: (b, 0, 0)),
            pl.BlockSpec((1, R, 1), lambda b, i: (b, i, 0)),
            pl.BlockSpec((1, 1, N), lambda b, i: (b, 0, 0)),
            pl.BlockSpec((1, 1, K + 1), lambda b, i: (b, 0, 0)),
            full((1, 1)),
        ],
        out_specs=pl.BlockSpec((1, R, D), lambda b, i: (b, i, 0)),
        out_shape=jax.ShapeDtypeStruct((B, N, D), x.dtype),
    )(x, x, mask_row, mask_col, cfit, scale2)
    return jnp.concatenate([y, y], axis=-1)


# speedup vs baseline: 1.0290x; 1.0290x over previous
import functools

import jax
import jax.numpy as jnp
import numpy as np
from jax.experimental import pallas as pl

_ROWS = 512
_K = 41
_CH = 16


def _cheb_consts():
    k = np.arange(_K)
    theta = (k + 0.5) * np.pi / _K
    xnodes = np.cos(theta)[None, :]
    j = np.arange(_K)[:, None]
    fit = (2.0 / _K) * np.cos(j * theta[None, :])
    fit[0, :] *= 0.5
    return xnodes.astype(np.float32), fit.astype(np.float32)

_XNODES, _FITM = _cheb_consts()


def _lipswish(t):
    return 0.909 * t * jax.nn.sigmoid(t)


def _dist2_block(xb, xf, hi):
    nb = jnp.sum(xb * xb, axis=1, keepdims=True)
    nf = jnp.sum(xf * xf, axis=1, keepdims=True)
    xb_aug = jnp.concatenate([xb * -2.0, nb, jnp.ones_like(nb)], axis=1)
    xf_aug = jnp.concatenate([xf, jnp.ones_like(nf), nf], axis=1)
    d16 = jax.lax.dot_general(
        xb_aug, xf_aug, (((1,), (1,)), ((), ())),
        preferred_element_type=jnp.float32, precision=hi)
    return jnp.maximum(d16 * 2.0, 0.0)


def _fit_kernel(
    xf_ref, W1c_ref, b1c_ref, We2_ref, b2c_ref, Wg_ref, bg_ref,
    Wc1_ref, b3c_ref, Wc2_ref, bc2_ref, xn_ref, fit_ref,
    cfit_ref,
):
    hi = jax.lax.Precision.HIGHEST
    lo = jax.lax.Precision.DEFAULT
    xf = xf_ref[0]
    dist2 = _dist2_block(xf, xf, hi)
    dmax = jnp.maximum(jnp.max(dist2), 1e-6)

    dn = (xn_ref[...] + 1.0) * (0.5 * dmax)
    m1 = _lipswish(W1c_ref[...] * dn + b1c_ref[...])
    m2 = jax.lax.dot_general(
        We2_ref[...], m1, (((0,), (0,)), ((), ())),
        preferred_element_type=jnp.float32, precision=lo)
    m2 = _lipswish(m2 + b2c_ref[...])
    gate = jax.nn.sigmoid(
        jax.lax.dot_general(
            Wg_ref[...], m2, (((0,), (0,)), ((), ())),
            preferred_element_type=jnp.float32, precision=lo)
        + bg_ref[...])
    h = _lipswish(
        jax.lax.dot_general(
            Wc1_ref[...], m2 * gate, (((0,), (0,)), ((), ())),
            preferred_element_type=jnp.float32, precision=lo)
        + b3c_ref[...])
    wn = jax.lax.dot_general(
        Wc2_ref[...], h, (((0,), (0,)), ((), ())),
        preferred_element_type=jnp.float32, precision=lo) + bc2_ref[...]

    c = jax.lax.dot_general(
        wn, fit_ref[...], (((1,), (1,)), ((), ())),
        preferred_element_type=jnp.float32, precision=hi)
    cfit_ref[0] = jnp.concatenate([c, jnp.full((1, 1), dmax)], axis=1)


def _egnn_block_kernel(
    xb_ref, xf_ref, mrow_ref, mcol_ref, cfit_ref, scale_ref,
    out_ref,
):
    R = xb_ref.shape[1]
    N = xf_ref.shape[1]
    hi = jax.lax.Precision.HIGHEST

    xb = xb_ref[0]
    xf = xf_ref[0]
    mrow = mrow_ref[0]
    mcol = mcol_ref[0]
    scale = scale_ref[0, 0]
    cs = [cfit_ref[0, 0, j] for j in range(_K)]
    dmax = cfit_ref[0, 0, _K]

    dist2 = _dist2_block(xb, xf, hi)

    xs = dist2 * (2.0 / dmax) - 1.0
    chunks = []
    for rc in range(0, R, _CH):
        xsc = xs[rc:rc + _CH]
        xs2c = xsc + xsc
        b1 = jnp.zeros_like(xsc)
        b2 = jnp.zeros_like(xsc)
        for j in range(_K - 1, 0, -1):
            b1, b2 = xs2c * b1 - b2 + cs[j], b1
        chunks.append(xsc * b1 - b2 + cs[0])
    w = jnp.concatenate(chunks, axis=0)
    w = jnp.clip(w, -2.0, 2.0)

    inv_norm = jax.lax.rsqrt(dist2 + 1e-8)
    a = w * mcol * (scale * inv_norm)
    r0 = pl.program_id(1) * R
    col_ids = jax.lax.broadcasted_iota(jnp.int32, (R, N), 1)
    row_ids = jax.lax.broadcasted_iota(jnp.int32, (R, N), 0) + r0
    a = jnp.where(col_ids == row_ids, 0.0, a)

    s = jnp.sum(a, axis=1, keepdims=True)
    t = jax.lax.dot_general(
        a, xf, (((1,), (0,)), ((), ())),
        preferred_element_type=jnp.float32, precision=hi)
    out_ref[0] = (xb + mrow * (s * xb - t)) * mrow


@functools.partial(jax.jit, static_argnames=())
def kernel(x, mask, We1, be1, We2, be2, Wg, bg, Wc1, bc1, Wc2, bc2, scale):
    B, N, D = x.shape
    M = We2.shape[0]
    H = Wc1.shape[1]
    R = _ROWS
    K = _K

    mask_row = mask.reshape(B, N, 1)
    mask_col = mask.reshape(B, 1, N)
    W1c = We1.reshape(M, 1)
    b1c = be1.reshape(M, 1)
    b2c = be2.reshape(M, 1)
    bg2 = bg.reshape(1, 1)
    b3c = bc1.reshape(H, 1)
    bc22 = bc2.reshape(1, 1)
    scale2 = scale.reshape(1, 1)
    xn = jnp.asarray(_XNODES)
    fitm = jnp.asarray(_FITM)

    fullb = lambda shape: pl.BlockSpec(shape, lambda b: (0,) * len(shape))
    cfit = pl.pallas_call(
        _fit_kernel,
        grid=(B,),
        in_specs=[
            pl.BlockSpec((1, N, D), lambda b: (b, 0, 0)),
            fullb((M, 1)), fullb((M, 1)), fullb((M, M)), fullb((M, 1)),
            fullb((M, 1)), fullb((1, 1)), fullb((M, H)), fullb((H, 1)),
            fullb((H, 1)), fullb((1, 1)), fullb((1, K)), fullb((K, K)),
        ],
        out_specs=pl.BlockSpec((1, 1, K + 1), lambda b: (b, 0, 0)),
        out_shape=jax.ShapeDtypeStruct((B, 1, K + 1), jnp.float32),
    )(x, W1c, b1c, We2, b2c, Wg, bg2, Wc1, b3c, Wc2, bc22, xn, fitm)

    grid = (B, N // R)
    full = lambda shape: pl.BlockSpec(shape, lambda b, i: (0,) * len(shape))
    y = pl.pallas_call(
        _egnn_block_kernel,
        grid=grid,
        in_specs=[
            pl.BlockSpec((1, R, D), lambda b, i: (b, i, 0)),
            pl.BlockSpec((1, N, D), lambda b, i: (b, 0, 0)),
            pl.BlockSpec((1, R, 1), lambda b, i: (b, i, 0)),
            pl.BlockSpec((1, 1, N), lambda b, i: (b, 0, 0)),
            pl.BlockSpec((1, 1, K + 1), lambda b, i: (b, 0, 0)),
            full((1, 1)),
        ],
        out_specs=pl.BlockSpec((1, R, D), lambda b, i: (b, i, 0)),
        out_shape=jax.ShapeDtypeStruct((B, N, D), x.dtype),
    )(x, x, mask_row, mask_col, cfit, scale2)
    return jnp.concatenate([y, y], axis=-1)


# speedup vs baseline: 1.0325x; 1.0035x over previous
import functools

import jax
import jax.numpy as jnp
import numpy as np
from jax.experimental import pallas as pl

_ROWS = 512
_K = 41
_CH = 32


def _cheb_consts():
    k = np.arange(_K)
    theta = (k + 0.5) * np.pi / _K
    xnodes = np.cos(theta)[None, :]
    j = np.arange(_K)[:, None]
    fit = (2.0 / _K) * np.cos(j * theta[None, :])
    fit[0, :] *= 0.5
    return xnodes.astype(np.float32), fit.astype(np.float32)

_XNODES, _FITM = _cheb_consts()


def _lipswish(t):
    return 0.909 * t * jax.nn.sigmoid(t)


def _dist2_block(xb, xf, hi):
    nb = jnp.sum(xb * xb, axis=1, keepdims=True)
    nf = jnp.sum(xf * xf, axis=1, keepdims=True)
    xb_aug = jnp.concatenate([xb * -2.0, nb, jnp.ones_like(nb)], axis=1)
    xf_aug = jnp.concatenate([xf, jnp.ones_like(nf), nf], axis=1)
    d16 = jax.lax.dot_general(
        xb_aug, xf_aug, (((1,), (1,)), ((), ())),
        preferred_element_type=jnp.float32, precision=hi)
    return jnp.maximum(d16 * 2.0, 0.0)


def _fit_kernel(
    xf_ref, W1c_ref, b1c_ref, We2_ref, b2c_ref, Wg_ref, bg_ref,
    Wc1_ref, b3c_ref, Wc2_ref, bc2_ref, xn_ref, fit_ref,
    cfit_ref,
):
    hi = jax.lax.Precision.HIGHEST
    lo = jax.lax.Precision.DEFAULT
    xf = xf_ref[0]
    dist2 = _dist2_block(xf, xf, hi)
    dmax = jnp.maximum(jnp.max(dist2), 1e-6)

    dn = (xn_ref[...] + 1.0) * (0.5 * dmax)
    m1 = _lipswish(W1c_ref[...] * dn + b1c_ref[...])
    m2 = jax.lax.dot_general(
        We2_ref[...], m1, (((0,), (0,)), ((), ())),
        preferred_element_type=jnp.float32, precision=lo)
    m2 = _lipswish(m2 + b2c_ref[...])
    gate = jax.nn.sigmoid(
        jax.lax.dot_general(
            Wg_ref[...], m2, (((0,), (0,)), ((), ())),
            preferred_element_type=jnp.float32, precision=lo)
        + bg_ref[...])
    h = _lipswish(
        jax.lax.dot_general(
            Wc1_ref[...], m2 * gate, (((0,), (0,)), ((), ())),
            preferred_element_type=jnp.float32, precision=lo)
        + b3c_ref[...])
    wn = jax.lax.dot_general(
        Wc2_ref[...], h, (((0,), (0,)), ((), ())),
        preferred_element_type=jnp.float32, precision=lo) + bc2_ref[...]

    c = jax.lax.dot_general(
        wn, fit_ref[...], (((1,), (1,)), ((), ())),
        preferred_element_type=jnp.float32, precision=hi)
    cfit_ref[0] = jnp.concatenate([c, jnp.full((1, 1), dmax)], axis=1)


def _egnn_block_kernel(
    xb_ref, xf_ref, mrow_ref, mcol_ref, cfit_ref, scale_ref,
    out_ref,
):
    R = xb_ref.shape[1]
    N = xf_ref.shape[1]
    hi = jax.lax.Precision.HIGHEST

    xb = xb_ref[0]
    xf = xf_ref[0]
    mrow = mrow_ref[0]
    mcol = mcol_ref[0]
    scale = scale_ref[0, 0]
    cs = [cfit_ref[0, 0, j] for j in range(_K)]
    dmax = cfit_ref[0, 0, _K]

    dist2 = _dist2_block(xb, xf, hi)

    xs = dist2 * (2.0 / dmax) - 1.0
    chunks = []
    for rc in range(0, R, _CH):
        xsc = xs[rc:rc + _CH]
        xs2c = xsc + xsc
        b1 = jnp.zeros_like(xsc)
        b2 = jnp.zeros_like(xsc)
        for j in range(_K - 1, 0, -1):
            b1, b2 = xs2c * b1 - b2 + cs[j], b1
        chunks.append(xsc * b1 - b2 + cs[0])
    w = jnp.concatenate(chunks, axis=0)
    w = jnp.clip(w, -2.0, 2.0)

    inv_norm = jax.lax.rsqrt(dist2 + 1e-8)
    a = w * mcol * (scale * inv_norm)
    r0 = pl.program_id(1) * R
    col_ids = jax.lax.broadcasted_iota(jnp.int32, (R, N), 1)
    row_ids = jax.lax.broadcasted_iota(jnp.int32, (R, N), 0) + r0
    a = jnp.where(col_ids == row_ids, 0.0, a)

    s = jnp.sum(a, axis=1, keepdims=True)
    t = jax.lax.dot_general(
        a, xf, (((1,), (0,)), ((), ())),
        preferred_element_type=jnp.float32, precision=hi)
    out_ref[0] = (xb + mrow * (s * xb - t)) * mrow


@functools.partial(jax.jit, static_argnames=())
def kernel(x, mask, We1, be1, We2, be2, Wg, bg, Wc1, bc1, Wc2, bc2, scale):
    B, N, D = x.shape
    M = We2.shape[0]
    H = Wc1.shape[1]
    R = _ROWS
    K = _K

    mask_row = mask.reshape(B, N, 1)
    mask_col = mask.reshape(B, 1, N)
    W1c = We1.reshape(M, 1)
    b1c = be1.reshape(M, 1)
    b2c = be2.reshape(M, 1)
    bg2 = bg.reshape(1, 1)
    b3c = bc1.reshape(H, 1)
    bc22 = bc2.reshape(1, 1)
    scale2 = scale.reshape(1, 1)
    xn = jnp.asarray(_XNODES)
    fitm = jnp.asarray(_FITM)

    fullb = lambda shape: pl.BlockSpec(shape, lambda b: (0,) * len(shape))
    cfit = pl.pallas_call(
        _fit_kernel,
        grid=(B,),
        in_specs=[
            pl.BlockSpec((1, N, D), lambda b: (b, 0, 0)),
            fullb((M, 1)), fullb((M, 1)), fullb((M, M)), fullb((M, 1)),
            fullb((M, 1)), fullb((1, 1)), fullb((M, H)), fullb((H, 1)),
            fullb((H, 1)), fullb((1, 1)), fullb((1, K)), fullb((K, K)),
        ],
        out_specs=pl.BlockSpec((1, 1, K + 1), lambda b: (b, 0, 0)),
        out_shape=jax.ShapeDtypeStruct((B, 1, K + 1), jnp.float32),
    )(x, W1c, b1c, We2, b2c, Wg, bg2, Wc1, b3c, Wc2, bc22, xn, fitm)

    grid = (B, N // R)
    full = lambda shape: pl.BlockSpec(shape, lambda b, i: (0,) * len(shape))
    y = pl.pallas_call(
        _egnn_block_kernel,
        grid=grid,
        in_specs=[
            pl.BlockSpec((1, R, D), lambda b, i: (b, i, 0)),
            pl.BlockSpec((1, N, D), lambda b, i: (b, 0, 0)),
            pl.BlockSpec((1, R, 1), lambda b, i: (b, i, 0)),
            pl.BlockSpec((1, 1, N), lambda b, i: (b, 0, 0)),
            pl.BlockSpec((1, 1, K + 1), lambda b, i: (b, 0, 0)),
            full((1, 1)),
        ],
        out_specs=pl.BlockSpec((1, R, D), lambda b, i: (b, i, 0)),
        out_shape=jax.ShapeDtypeStruct((B, N, D), x.dtype),
    )(x, x, mask_row, mask_col, cfit, scale2)
    return jnp.concatenate([y, y], axis=-1)
